# Initial kernel scaffold; baseline (speedup 1.0000x reference)
import jax
import jax.numpy as jnp
from jax.experimental import pallas as pl


def kernel(sv, W_ih, W_hh, b_ih, b_hh):
    raise NotImplementedError("write your pallas kernel here")


# speedup vs baseline: 4.2259x; 4.2259x over previous
import jax
import jax.numpy as jnp
from jax import lax
from jax.experimental import pallas as pl
from jax.experimental.pallas import tpu as pltpu

T_CHUNK = 64


def _lstm_body(sv_ref, wih_ref, whh_ref, bias_ref, out_ref, xg_ref, h_ref, c_ref):
    i = pl.program_id(0)
    nb = sv_ref.shape[0] // T_CHUNK
    hdim = h_ref.shape[1]

    @pl.when(i == 0)
    def _init():
        h_ref[...] = jnp.zeros_like(h_ref)
        c_ref[...] = jnp.zeros_like(c_ref)

    xg_ref[...] = (
        jnp.dot(sv_ref[...], wih_ref[...], preferred_element_type=jnp.float32)
        + bias_ref[...]
    )

    def step(t, carry):
        h, c = carry
        g = xg_ref[pl.ds(t * nb, nb), :] + jnp.dot(
            h, whh_ref[...], preferred_element_type=jnp.float32
        )
        gi = jax.nn.sigmoid(g[:, 0 * hdim : 1 * hdim])
        gf = jax.nn.sigmoid(g[:, 1 * hdim : 2 * hdim])
        gg = jnp.tanh(g[:, 2 * hdim : 3 * hdim])
        go = jax.nn.sigmoid(g[:, 3 * hdim : 4 * hdim])
        c_new = gf * c + gi * gg
        h_new = go * jnp.tanh(c_new)
        return h_new, c_new

    h, c = lax.fori_loop(0, T_CHUNK, step, (h_ref[...], c_ref[...]))
    h_ref[...] = h
    c_ref[...] = c

    @pl.when(i == pl.num_programs(0) - 1)
    def _emit():
        out_ref[...] = h


def kernel(sv, W_ih, W_hh, b_ih, b_hh):
    b, l, d = sv.shape
    h4 = W_ih.shape[0]
    hdim = W_hh.shape[1]
    nchunk = l // T_CHUNK

    sv_tm = jnp.swapaxes(sv, 0, 1).reshape(l * b, d)
    wih_t = W_ih.T
    whh_t = W_hh.T
    bias = (b_ih + b_hh).reshape(1, h4)

    return pl.pallas_call(
        _lstm_body,
        grid=(nchunk,),
        in_specs=[
            pl.BlockSpec((T_CHUNK * b, d), lambda i: (i, 0)),
            pl.BlockSpec((d, h4), lambda i: (0, 0)),
            pl.BlockSpec((hdim, h4), lambda i: (0, 0)),
            pl.BlockSpec((1, h4), lambda i: (0, 0)),
        ],
        out_specs=pl.BlockSpec((b, hdim), lambda i: (0, 0)),
        out_shape=jax.ShapeDtypeStruct((b, hdim), jnp.float32),
        scratch_shapes=[
            pltpu.VMEM((T_CHUNK * b, h4), jnp.float32),
            pltpu.VMEM((b, hdim), jnp.float32),
            pltpu.VMEM((b, hdim), jnp.float32),
        ],
    )(sv_tm, wih_t, whh_t, bias)
